# baseline (speedup 1.0000x reference)
import jax
import jax.numpy as jnp
from jax import lax
from jax.experimental import pallas as pl
from jax.experimental.pallas import tpu as pltpu

_ROWS = 16384
_COLS = 4096
_BM = 256
_NB = _ROWS // _BM
_RIN = 2
_ROUT = 2
_K = 10
_NRING = _NB - _K - _ROUT
_STASH0 = _NRING + _ROUT


def _body(x_hbm, o_hbm, inb, outb, stash, acc_ref,
          sem_in, sem_out, sem_stash):
    p = pl.program_id(0)
    i = pl.program_id(1)

    @pl.when((p == 0) & (i == 0))
    def _prime():
        for b in range(_RIN):
            pltpu.make_async_copy(
                x_hbm.at[pl.ds(b * _BM, _BM)], inb.at[b], sem_in.at[b]
            ).start()

    @pl.when(p == 0)
    def _reduce():
        @pl.when(i < _NRING)
        def _from_ring():
            s = lax.rem(i, _RIN)
            pltpu.make_async_copy(
                x_hbm.at[pl.ds(i * _BM, _BM)], inb.at[s], sem_in.at[s]
            ).wait()
            _accum(acc_ref, i, inb[s])

        @pl.when((i >= _NRING) & (i < _STASH0))
        def _from_outb():
            j = i - _NRING
            pltpu.make_async_copy(
                x_hbm.at[pl.ds(i * _BM, _BM)], outb.at[j], sem_out.at[j]
            ).wait()
            _accum(acc_ref, i, outb[j])

        @pl.when(i >= _STASH0)
        def _from_stash():
            j = i - _STASH0
            pltpu.make_async_copy(
                x_hbm.at[pl.ds(i * _BM, _BM)], stash.at[j], sem_stash.at[j]
            ).wait()
            _accum(acc_ref, i, stash[j])

        nxt = i + _RIN

        @pl.when(nxt < _NRING)
        def _refill_ring():
            s = lax.rem(nxt, _RIN)
            pltpu.make_async_copy(
                x_hbm.at[pl.ds(nxt * _BM, _BM)], inb.at[s], sem_in.at[s]
            ).start()

        @pl.when((nxt >= _NRING) & (nxt < _STASH0))
        def _refill_outb():
            j = nxt - _NRING
            pltpu.make_async_copy(
                x_hbm.at[pl.ds(nxt * _BM, _BM)], outb.at[j], sem_out.at[j]
            ).start()

        @pl.when((nxt >= _STASH0) & (nxt < _NB))
        def _refill_stash():
            j = nxt - _STASH0
            pltpu.make_async_copy(
                x_hbm.at[pl.ds(nxt * _BM, _BM)], stash.at[j],
                sem_stash.at[j]
            ).start()

        @pl.when(i >= _NB - _RIN)
        def _prefetch_b():
            b = i - (_NB - _RIN)
            pltpu.make_async_copy(
                x_hbm.at[pl.ds(b * _BM, _BM)], inb.at[b], sem_in.at[b]
            ).start()

    @pl.when(p == 1)
    def _rescale():
        mn = acc_ref[0]
        scale = 1.0 / (acc_ref[1] - mn)
        o = lax.rem(i, _ROUT)
        blk = jnp.where(i < _ROUT, i + _NRING,
                        jnp.where(i < _STASH0, i - _ROUT, i))

        @pl.when(i >= _ROUT)
        def _wait_prev_out():
            pltpu.make_async_copy(
                outb.at[o], o_hbm.at[pl.ds(0, _BM)], sem_out.at[o]
            ).wait()

        @pl.when(i < _ROUT)
        def _outb_path():
            outb[o] = (outb[o] - mn) * scale

        @pl.when((i >= _ROUT) & (i < _STASH0))
        def _ring_path():
            b = i - _ROUT
            s = lax.rem(b, _RIN)
            pltpu.make_async_copy(
                x_hbm.at[pl.ds(b * _BM, _BM)], inb.at[s], sem_in.at[s]
            ).wait()
            outb[o] = (inb[s] - mn) * scale

        @pl.when(i >= _STASH0)
        def _stash_path():
            outb[o] = (stash[i - _STASH0] - mn) * scale

        pltpu.make_async_copy(
            outb.at[o], o_hbm.at[pl.ds(blk * _BM, _BM)], sem_out.at[o]
        ).start()

        nb = i - _ROUT + _RIN

        @pl.when((i >= _ROUT) & (nb < _NRING))
        def _refill_b():
            s = lax.rem(nb, _RIN)
            pltpu.make_async_copy(
                x_hbm.at[pl.ds(nb * _BM, _BM)], inb.at[s], sem_in.at[s]
            ).start()

        @pl.when(i == _NB - 1)
        def _drain():
            for o2 in range(_ROUT):
                pltpu.make_async_copy(
                    outb.at[o2], o_hbm.at[pl.ds(0, _BM)], sem_out.at[o2]
                ).wait()


def _accum(acc_ref, i, v):
    bmn = jnp.min(v)
    bmx = jnp.max(v)

    @pl.when(i == 0)
    def _init():
        acc_ref[0] = bmn
        acc_ref[1] = bmx

    @pl.when(i > 0)
    def _acc():
        acc_ref[0] = jnp.minimum(acc_ref[0], bmn)
        acc_ref[1] = jnp.maximum(acc_ref[1], bmx)


def kernel(inp):
    return pl.pallas_call(
        _body,
        grid=(2, _NB),
        in_specs=[pl.BlockSpec(memory_space=pl.ANY)],
        out_specs=pl.BlockSpec(memory_space=pl.ANY),
        out_shape=jax.ShapeDtypeStruct((_ROWS, _COLS), jnp.float32),
        scratch_shapes=[
            pltpu.VMEM((_RIN, _BM, _COLS), jnp.float32),
            pltpu.VMEM((_ROUT, _BM, _COLS), jnp.float32),
            pltpu.VMEM((_K, _BM, _COLS), jnp.float32),
            pltpu.SMEM((2,), jnp.float32),
            pltpu.SemaphoreType.DMA((_RIN,)),
            pltpu.SemaphoreType.DMA((_ROUT,)),
            pltpu.SemaphoreType.DMA((_K,)),
        ],
    )(inp)


# speedup vs baseline: 1.1613x; 1.1613x over previous
import jax
import jax.numpy as jnp
from jax import lax
from jax.experimental import pallas as pl
from jax.experimental.pallas import tpu as pltpu

_ROWS = 16384
_COLS = 4096
_BM = 256
_NB = _ROWS // _BM
_RIN = 3
_ROUT = 3
_K = 8
_NRING = _NB - _K - _ROUT
_STASH0 = _NRING + _ROUT


def _body(x_hbm, o_hbm, inb, outb, stash, acc_ref,
          sem_in, sem_out, sem_stash):
    p = pl.program_id(0)
    i = pl.program_id(1)

    @pl.when((p == 0) & (i == 0))
    def _prime():
        for b in range(_RIN):
            pltpu.make_async_copy(
                x_hbm.at[pl.ds(b * _BM, _BM)], inb.at[b], sem_in.at[b]
            ).start()

    @pl.when(p == 0)
    def _reduce():
        @pl.when(i < _NRING)
        def _from_ring():
            s = lax.rem(i, _RIN)
            pltpu.make_async_copy(
                x_hbm.at[pl.ds(i * _BM, _BM)], inb.at[s], sem_in.at[s]
            ).wait()
            _accum(acc_ref, i, inb[s])

        @pl.when((i >= _NRING) & (i < _STASH0))
        def _from_outb():
            j = i - _NRING
            pltpu.make_async_copy(
                x_hbm.at[pl.ds(i * _BM, _BM)], outb.at[j], sem_out.at[j]
            ).wait()
            _accum(acc_ref, i, outb[j])

        @pl.when(i >= _STASH0)
        def _from_stash():
            j = i - _STASH0
            pltpu.make_async_copy(
                x_hbm.at[pl.ds(i * _BM, _BM)], stash.at[j], sem_stash.at[j]
            ).wait()
            _accum(acc_ref, i, stash[j])

        nxt = i + _RIN

        @pl.when(nxt < _NRING)
        def _refill_ring():
            s = lax.rem(nxt, _RIN)
            pltpu.make_async_copy(
                x_hbm.at[pl.ds(nxt * _BM, _BM)], inb.at[s], sem_in.at[s]
            ).start()

        @pl.when((nxt >= _NRING) & (nxt < _STASH0))
        def _refill_outb():
            j = nxt - _NRING
            pltpu.make_async_copy(
                x_hbm.at[pl.ds(nxt * _BM, _BM)], outb.at[j], sem_out.at[j]
            ).start()

        @pl.when((nxt >= _STASH0) & (nxt < _NB))
        def _refill_stash():
            j = nxt - _STASH0
            pltpu.make_async_copy(
                x_hbm.at[pl.ds(nxt * _BM, _BM)], stash.at[j],
                sem_stash.at[j]
            ).start()

        @pl.when(i >= _NB - _RIN)
        def _prefetch_b():
            b = i - (_NB - _RIN)
            pltpu.make_async_copy(
                x_hbm.at[pl.ds(b * _BM, _BM)], inb.at[b], sem_in.at[b]
            ).start()

    @pl.when(p == 1)
    def _rescale():
        mn = acc_ref[0]
        scale = 1.0 / (acc_ref[1] - mn)
        o = lax.rem(i, _ROUT)
        blk = jnp.where(i < _ROUT, i + _NRING,
                        jnp.where(i < _STASH0, i - _ROUT, i))

        @pl.when(i >= _ROUT)
        def _wait_prev_out():
            pltpu.make_async_copy(
                outb.at[o], o_hbm.at[pl.ds(0, _BM)], sem_out.at[o]
            ).wait()

        @pl.when(i < _ROUT)
        def _outb_path():
            outb[o] = (outb[o] - mn) * scale

        @pl.when((i >= _ROUT) & (i < _STASH0))
        def _ring_path():
            b = i - _ROUT
            s = lax.rem(b, _RIN)
            pltpu.make_async_copy(
                x_hbm.at[pl.ds(b * _BM, _BM)], inb.at[s], sem_in.at[s]
            ).wait()
            outb[o] = (inb[s] - mn) * scale

        @pl.when(i >= _STASH0)
        def _stash_path():
            outb[o] = (stash[i - _STASH0] - mn) * scale

        pltpu.make_async_copy(
            outb.at[o], o_hbm.at[pl.ds(blk * _BM, _BM)], sem_out.at[o]
        ).start()

        nb = i - _ROUT + _RIN

        @pl.when((i >= _ROUT) & (nb < _NRING))
        def _refill_b():
            s = lax.rem(nb, _RIN)
            pltpu.make_async_copy(
                x_hbm.at[pl.ds(nb * _BM, _BM)], inb.at[s], sem_in.at[s]
            ).start()

        @pl.when(i == _NB - 1)
        def _drain():
            for o2 in range(_ROUT):
                pltpu.make_async_copy(
                    outb.at[o2], o_hbm.at[pl.ds(0, _BM)], sem_out.at[o2]
                ).wait()


def _accum(acc_ref, i, v):
    bmn = jnp.min(v)
    bmx = jnp.max(v)

    @pl.when(i == 0)
    def _init():
        acc_ref[0] = bmn
        acc_ref[1] = bmx

    @pl.when(i > 0)
    def _acc():
        acc_ref[0] = jnp.minimum(acc_ref[0], bmn)
        acc_ref[1] = jnp.maximum(acc_ref[1], bmx)


def kernel(inp):
    return pl.pallas_call(
        _body,
        grid=(2, _NB),
        in_specs=[pl.BlockSpec(memory_space=pl.ANY)],
        out_specs=pl.BlockSpec(memory_space=pl.ANY),
        out_shape=jax.ShapeDtypeStruct((_ROWS, _COLS), jnp.float32),
        scratch_shapes=[
            pltpu.VMEM((_RIN, _BM, _COLS), jnp.float32),
            pltpu.VMEM((_ROUT, _BM, _COLS), jnp.float32),
            pltpu.VMEM((_K, _BM, _COLS), jnp.float32),
            pltpu.SMEM((2,), jnp.float32),
            pltpu.SemaphoreType.DMA((_RIN,)),
            pltpu.SemaphoreType.DMA((_ROUT,)),
            pltpu.SemaphoreType.DMA((_K,)),
        ],
    )(inp)
